# baseline (speedup 1.0000x reference)
import functools

import jax
import jax.numpy as jnp
from jax import lax
from jax.experimental import pallas as pl
from jax.experimental.pallas import tpu as pltpu
from jax.experimental.pallas import tpu_sc as plsc

_VOCAB = 100000
_D = 128
_B = 4096
_L = 200
_OFFSET = 0.5

_COLS = 2048
_NBLK = 25
_HALF = _NBLK * _COLS


def _matvec_body(emb_lo_ref, emb_hi_ref, fc_ref, out_ref):
    w = jnp.clip(fc_ref[...], 0.0, 1.0)
    dims = (((1,), (1,)), ((), ()))

    def half_bits(e_ref):
        e = jnp.clip(e_ref[...], 0.0, 1.0)
        r = lax.dot_general(w, e, dims,
                            preferred_element_type=jnp.float32)
        b16 = lax.bitcast_convert_type(r.astype(jnp.bfloat16), jnp.uint16)
        return b16.astype(jnp.int32)

    packed = half_bits(emb_lo_ref) | (half_bits(emb_hi_ref) << 16)
    out_ref[...] = packed.reshape(1, 1, _COLS)


def _compute_s(emb_weight, fc_weight):
    s3 = pl.pallas_call(
        _matvec_body,
        grid=(_NBLK,),
        in_specs=[
            pl.BlockSpec((_COLS, _D), lambda i: (i, 0)),
            pl.BlockSpec((_COLS, _D),
                         lambda i: (jnp.minimum(i + _NBLK, 2 * _NBLK - 2), 0)),
            pl.BlockSpec((1, _D), lambda i: (0, 0)),
        ],
        out_specs=pl.BlockSpec((1, 1, _COLS), lambda i: (i, 0, 0)),
        out_shape=jax.ShapeDtypeStruct((_NBLK, 1, _COLS), jnp.int32),
    )(emb_weight, emb_weight, fc_weight)
    return s3.reshape(_HALF)


_NW = 32
_BPW = _B // _NW
_NCHUNK = _BPW // 16
_IPW = _L * _BPW


def _sc_body(text_t, s_vec, out, idx_v, s_t, out_v, sem, sem2):
    wid = lax.axis_index("s") * 2 + lax.axis_index("c")
    base = wid * _BPW
    cp1 = pltpu.make_async_copy(text_t.at[:, pl.ds(base, _BPW)], idx_v, sem)
    cp2 = pltpu.make_async_copy(s_vec, s_t, sem2)
    cp1.start()
    cp2.start()
    cp1.wait()
    cp2.wait()

    himask = jnp.full((16,), -65536, jnp.int32)

    init = tuple(jnp.zeros((16,), jnp.float32) for _ in range(_NCHUNK))

    @plsc.parallel_loop(0, _L, unroll=4, carry=init)
    def accs(l, accs):
        new = []
        for c in range(_NCHUNK):
            iv = idx_v[l, pl.ds(c * 16, 16)]
            in_hi = iv >= _HALF
            slot = jnp.where(in_hi, iv - _HALF, iv)
            g = plsc.load_gather(s_t, [slot])
            bits = jnp.where(in_hi, g & himask, g << 16)
            new.append(accs[c] + plsc.bitcast(bits, jnp.float32))
        return tuple(new)
    for c in range(_NCHUNK):
        out_v[pl.ds(c * 16, 16)] = accs[c] * (1.0 / _L) - _OFFSET
    pltpu.sync_copy(out_v, out.at[pl.ds(base, _BPW)])


@functools.cache
def _sc_pool():
    return functools.partial(
        pl.kernel,
        mesh=plsc.VectorSubcoreMesh(core_axis_name="c", subcore_axis_name="s"),
        compiler_params=pltpu.CompilerParams(needs_layout_passes=False),
        out_type=jax.ShapeDtypeStruct((_B,), jnp.float32),
        scratch_types=[
            pltpu.VMEM((_L, _BPW), jnp.int32),
            pltpu.VMEM((_HALF,), jnp.int32),
            pltpu.VMEM((_BPW,), jnp.float32),
            pltpu.SemaphoreType.DMA,
            pltpu.SemaphoreType.DMA,
        ],
    )(_sc_body)


def kernel(text, emb_weight, fc_weight):
    s = _compute_s(emb_weight, fc_weight)
    return _sc_pool()(text.astype(jnp.int32).T, s)


# speedup vs baseline: 1.1670x; 1.1670x over previous
import functools

import jax
import jax.numpy as jnp
from jax import lax
from jax.experimental import pallas as pl
from jax.experimental.pallas import tpu as pltpu
from jax.experimental.pallas import tpu_sc as plsc

_VOCAB = 100000
_D = 128
_B = 4096
_L = 200
_OFFSET = 0.5

_COLS = 4096
_NBLK = 13
_HALF = _NBLK * _COLS


def _matvec_body(emb_lo_ref, emb_hi_ref, fc_ref, out_ref):
    w = jnp.clip(fc_ref[...], 0.0, 1.0)
    dims = (((1,), (1,)), ((), ()))

    def half_bits(e_ref):
        e = jnp.clip(e_ref[...], 0.0, 1.0)
        r = lax.dot_general(w, e, dims,
                            preferred_element_type=jnp.float32)
        b16 = lax.bitcast_convert_type(r.astype(jnp.bfloat16), jnp.uint16)
        return b16.astype(jnp.int32)

    packed = half_bits(emb_lo_ref) | (half_bits(emb_hi_ref) << 16)
    out_ref[...] = packed.reshape(1, 1, _COLS)


def _compute_s(emb_weight, fc_weight):
    s3 = pl.pallas_call(
        _matvec_body,
        grid=(_NBLK,),
        in_specs=[
            pl.BlockSpec((_COLS, _D), lambda i: (i, 0)),
            pl.BlockSpec((_COLS, _D),
                         lambda i: (jnp.minimum(i + _NBLK, 24), 0)),
            pl.BlockSpec((1, _D), lambda i: (0, 0)),
        ],
        out_specs=pl.BlockSpec((1, 1, _COLS), lambda i: (i, 0, 0)),
        out_shape=jax.ShapeDtypeStruct((_NBLK, 1, _COLS), jnp.int32),
    )(emb_weight, emb_weight, fc_weight)
    return s3.reshape(_HALF)


_NW = 32
_BPW = _B // _NW
_NCHUNK = _BPW // 16
_IPW = _L * _BPW


def _sc_body(text_t, s_vec, out, idx_v, s_t, out_v, sem, sem2):
    wid = lax.axis_index("s") * 2 + lax.axis_index("c")
    base = wid * _BPW
    cp1 = pltpu.make_async_copy(text_t.at[:, pl.ds(base, _BPW)], idx_v, sem)
    cp2 = pltpu.make_async_copy(s_vec, s_t, sem2)
    cp1.start()
    cp2.start()
    cp1.wait()
    cp2.wait()

    himask = jnp.full((16,), -65536, jnp.int32)

    def body(l, accs):
        new = []
        for c in range(_NCHUNK):
            iv = idx_v[l, pl.ds(c * 16, 16)]
            in_hi = iv >= _HALF
            slot = jnp.where(in_hi, iv - _HALF, iv)
            g = plsc.load_gather(s_t, [slot])
            bits = jnp.where(in_hi, g & himask, g << 16)
            new.append(accs[c] + plsc.bitcast(bits, jnp.float32))
        return tuple(new)

    accs = lax.fori_loop(
        0, _L, body,
        tuple(jnp.zeros((16,), jnp.float32) for _ in range(_NCHUNK)))
    for c in range(_NCHUNK):
        out_v[pl.ds(c * 16, 16)] = accs[c] * (1.0 / _L) - _OFFSET
    pltpu.sync_copy(out_v, out.at[pl.ds(base, _BPW)])


@functools.cache
def _sc_pool():
    return functools.partial(
        pl.kernel,
        mesh=plsc.VectorSubcoreMesh(core_axis_name="c", subcore_axis_name="s"),
        compiler_params=pltpu.CompilerParams(needs_layout_passes=False),
        out_type=jax.ShapeDtypeStruct((_B,), jnp.float32),
        scratch_types=[
            pltpu.VMEM((_L, _BPW), jnp.int32),
            pltpu.VMEM((_HALF,), jnp.int32),
            pltpu.VMEM((_BPW,), jnp.float32),
            pltpu.SemaphoreType.DMA,
            pltpu.SemaphoreType.DMA,
        ],
    )(_sc_body)


def kernel(text, emb_weight, fc_weight):
    s = _compute_s(emb_weight, fc_weight)
    return _sc_pool()(text.astype(jnp.int32).T, s)


# speedup vs baseline: 1.2247x; 1.0494x over previous
import functools

import jax
import jax.numpy as jnp
from jax import lax
from jax.experimental import pallas as pl
from jax.experimental.pallas import tpu as pltpu
from jax.experimental.pallas import tpu_sc as plsc

_VOCAB = 100000
_D = 128
_B = 4096
_L = 200
_OFFSET = 0.5

_COLS = 8192
_NBLK = 7
_HALF = _NBLK * _COLS


def _matvec_body(emb_lo_ref, emb_hi_ref, fc_ref, out_ref):
    w = jnp.clip(fc_ref[...], 0.0, 1.0)
    dims = (((1,), (1,)), ((), ()))

    def half_bits(e_ref):
        e = jnp.clip(e_ref[...], 0.0, 1.0)
        r = lax.dot_general(w, e, dims,
                            preferred_element_type=jnp.float32)
        b16 = lax.bitcast_convert_type(r.astype(jnp.bfloat16), jnp.uint16)
        return b16.astype(jnp.int32)

    packed = half_bits(emb_lo_ref) | (half_bits(emb_hi_ref) << 16)
    out_ref[...] = packed.reshape(1, 1, _COLS)


def _compute_s(emb_weight, fc_weight):
    s3 = pl.pallas_call(
        _matvec_body,
        grid=(_NBLK,),
        in_specs=[
            pl.BlockSpec((_COLS, _D), lambda i: (i, 0)),
            pl.BlockSpec((_COLS, _D),
                         lambda i: (jnp.minimum(i + _NBLK, 12), 0)),
            pl.BlockSpec((1, _D), lambda i: (0, 0)),
        ],
        out_specs=pl.BlockSpec((1, 1, _COLS), lambda i: (i, 0, 0)),
        out_shape=jax.ShapeDtypeStruct((_NBLK, 1, _COLS), jnp.int32),
    )(emb_weight, emb_weight, fc_weight)
    return s3.reshape(_HALF)


_NW = 32
_BPW = _B // _NW
_NCHUNK = _BPW // 16
_IPW = _L * _BPW


def _sc_body(text_t, s_vec, out, idx_v, s_t, out_v, sem, sem2):
    wid = lax.axis_index("s") * 2 + lax.axis_index("c")
    base = wid * _BPW
    cp1 = pltpu.make_async_copy(text_t.at[:, pl.ds(base, _BPW)], idx_v, sem)
    cp2 = pltpu.make_async_copy(s_vec, s_t, sem2)
    cp1.start()
    cp2.start()
    cp1.wait()
    cp2.wait()

    himask = jnp.full((16,), -65536, jnp.int32)

    def body(l, accs):
        new = []
        for c in range(_NCHUNK):
            iv = idx_v[l, pl.ds(c * 16, 16)]
            in_hi = iv >= _HALF
            slot = jnp.where(in_hi, iv - _HALF, iv)
            g = plsc.load_gather(s_t, [slot])
            bits = jnp.where(in_hi, g & himask, g << 16)
            new.append(accs[c] + plsc.bitcast(bits, jnp.float32))
        return tuple(new)

    accs = lax.fori_loop(
        0, _L, body,
        tuple(jnp.zeros((16,), jnp.float32) for _ in range(_NCHUNK)))
    for c in range(_NCHUNK):
        out_v[pl.ds(c * 16, 16)] = accs[c] * (1.0 / _L) - _OFFSET
    pltpu.sync_copy(out_v, out.at[pl.ds(base, _BPW)])


@functools.cache
def _sc_pool():
    return functools.partial(
        pl.kernel,
        mesh=plsc.VectorSubcoreMesh(core_axis_name="c", subcore_axis_name="s"),
        compiler_params=pltpu.CompilerParams(needs_layout_passes=False),
        out_type=jax.ShapeDtypeStruct((_B,), jnp.float32),
        scratch_types=[
            pltpu.VMEM((_L, _BPW), jnp.int32),
            pltpu.VMEM((_HALF,), jnp.int32),
            pltpu.VMEM((_BPW,), jnp.float32),
            pltpu.SemaphoreType.DMA,
            pltpu.SemaphoreType.DMA,
        ],
    )(_sc_body)


def kernel(text, emb_weight, fc_weight):
    s = _compute_s(emb_weight, fc_weight)
    return _sc_pool()(text.astype(jnp.int32).T, s)
